# Initial kernel scaffold; baseline (speedup 1.0000x reference)
import jax
import jax.numpy as jnp
from jax.experimental import pallas as pl


def kernel(x, y, visible, img_label, memory):
    raise NotImplementedError("write your pallas kernel here")


# speedup vs baseline: 1.4772x; 1.4772x over previous
import functools

import jax
import jax.numpy as jnp
from jax.experimental import pallas as pl

NB_CLASSES = 12
NUM_POS = 768
SFD = NUM_POS // NB_CLASSES
N_NEG = 4
B, D, M = 32, 256, 8192

N_BLK = 1024
GRID = M // N_BLK


def _body(t_ref, noise_ref, lbl_ref, mem_ref, sim_ref, nsim_ref, oh_ref):
    mem = mem_ref[...]
    sim_ref[...] = jax.lax.dot_general(
        t_ref[...], mem, (((1,), (1,)), ((), ())),
        preferred_element_type=jnp.float32)

    @pl.when(pl.program_id(0) == 0)
    def _():
        nsim_ref[...] = jax.lax.dot_general(
            noise_ref[...], mem[:NUM_POS], (((1,), (1,)), ((), ())),
            preferred_element_type=jnp.float32)
        lbl = lbl_ref[...]
        classes = jax.lax.broadcasted_iota(jnp.int32, (B, NB_CLASSES), 1)
        eq = (lbl == classes).astype(jnp.float32)
        cnt = jnp.sum(eq, axis=0, keepdims=True)
        oh_ref[...] = eq / jnp.maximum(cnt, 1.0)


@jax.jit
def kernel(x, y, visible, img_label, memory):
    t = x[:, :SFD, :].reshape(B * SFD, D)
    noise = x[:, SFD:, :].reshape(B * N_NEG, D)
    lbl = img_label.astype(jnp.int32).reshape(B, 1)

    sim, nsim, oh = pl.pallas_call(
        _body,
        grid=(GRID,),
        in_specs=[
            pl.BlockSpec((B * SFD, D), lambda j: (0, 0)),
            pl.BlockSpec((B * N_NEG, D), lambda j: (0, 0)),
            pl.BlockSpec((B, 1), lambda j: (0, 0)),
            pl.BlockSpec((N_BLK, D), lambda j: (j, 0)),
        ],
        out_specs=[
            pl.BlockSpec((B * SFD, N_BLK), lambda j: (0, j)),
            pl.BlockSpec((B * N_NEG, NUM_POS), lambda j: (0, 0)),
            pl.BlockSpec((B, NB_CLASSES), lambda j: (0, 0)),
        ],
        out_shape=[
            jax.ShapeDtypeStruct((B * SFD, M), jnp.float32),
            jax.ShapeDtypeStruct((B * N_NEG, NUM_POS), jnp.float32),
            jax.ShapeDtypeStruct((B, NB_CLASSES), jnp.float32),
        ],
    )(t, noise, lbl, memory)

    return (sim.reshape(B, SFD, M), y, nsim.reshape(B, N_NEG, NUM_POS), oh)
